# Initial kernel scaffold; baseline (speedup 1.0000x reference)
import jax
import jax.numpy as jnp
from jax.experimental import pallas as pl


def kernel(x, edge_index, W, b):
    raise NotImplementedError("write your pallas kernel here")


# speedup vs baseline: 34.3090x; 34.3090x over previous
import functools

import jax
import jax.numpy as jnp
from jax import lax
from jax.experimental import pallas as pl
from jax.experimental.pallas import tpu as pltpu
from jax.experimental.pallas import tpu_sc as plsc

N = 10000
D = 64
E = 640000
NC = 2
NS = 16
NW = NC * NS

CH = 128
NROW = E // CH
RPW = NROW // NW
RPW_LAST = NROW - (NW - 1) * RPW
KSUB = 4

STRIPE = 640
STRIPE_LAST = N - (NS - 1) * STRIPE


def _stripe_2hop(s, src_at, mid_at, dst_at):
    base = s * STRIPE

    @pl.when(s < NS - 1)
    def _():
        pltpu.sync_copy(src_at(base, STRIPE), mid_at(STRIPE))
        pltpu.sync_copy(mid_at(STRIPE), dst_at(base, STRIPE))

    @pl.when(s == NS - 1)
    def _():
        pltpu.sync_copy(src_at(base, STRIPE_LAST), mid_at(STRIPE_LAST))
        pltpu.sync_copy(mid_at(STRIPE_LAST), dst_at(base, STRIPE_LAST))


def _deg_body(dst_hbm, ones_hbm, zeros1_hbm, deg_out0, deg_out1,
              deg_sh, idx_v, ones_v, zbuf, sem):
    c = lax.axis_index("c")
    s = lax.axis_index("s")
    wid = c * NS + s

    _stripe_2hop(s, lambda b, n: zeros1_hbm.at[pl.ds(0, n)],
                 lambda n: zbuf.at[pl.ds(0, n)],
                 lambda b, n: deg_sh.at[pl.ds(b, n)])
    pltpu.sync_copy(ones_hbm, ones_v)
    plsc.subcore_barrier()

    row0 = wid * RPW
    nrows = jnp.where(wid == NW - 1, RPW_LAST, RPW)

    def chunk(k, carry):
        pltpu.sync_copy(dst_hbm.at[pl.ds(row0 + k * KSUB, KSUB)], idx_v)
        for j in range(KSUB):
            pltpu.sync_copy(ones_v, deg_sh.at[idx_v.at[j]], add=True)
        return carry

    lax.fori_loop(0, nrows // KSUB, chunk, 0)
    plsc.subcore_barrier()

    @pl.when(c == 0)
    def _():
        _stripe_2hop(s, lambda b, n: deg_sh.at[pl.ds(b, n)],
                     lambda n: zbuf.at[pl.ds(0, n)],
                     lambda b, n: deg_out0.at[pl.ds(b, n)])

    @pl.when(c == 1)
    def _():
        _stripe_2hop(s, lambda b, n: deg_sh.at[pl.ds(b, n)],
                     lambda n: zbuf.at[pl.ds(0, n)],
                     lambda b, n: deg_out1.at[pl.ds(b, n)])


def _deg_call(dst_r, ones_c, zeros1):
    mesh = plsc.VectorSubcoreMesh(core_axis_name="c", subcore_axis_name="s")
    return pl.kernel(
        _deg_body,
        out_type=[jax.ShapeDtypeStruct((N,), jnp.float32),
                  jax.ShapeDtypeStruct((N,), jnp.float32)],
        mesh=mesh,
        scratch_types=[
            pltpu.VMEM_SHARED((N,), jnp.float32),
            pltpu.VMEM((KSUB, CH), jnp.int32),
            pltpu.VMEM((CH,), jnp.float32),
            pltpu.VMEM((STRIPE,), jnp.float32),
            pltpu.SemaphoreType.DMA,
        ],
    )(dst_r, ones_c, zeros1)


def _agg_body(g_hbm, src_hbm, dst_hbm, zeros2_hbm, s_out0, s_out1,
              s_sh, idx_s, idx_d, rows_v, zbuf, sem):
    c = lax.axis_index("c")
    s = lax.axis_index("s")
    wid = c * NS + s

    _stripe_2hop(s, lambda b, n: zeros2_hbm.at[pl.ds(0, n)],
                 lambda n: zbuf.at[pl.ds(0, n)],
                 lambda b, n: s_sh.at[pl.ds(b, n)])
    plsc.subcore_barrier()

    row0 = wid * RPW
    nrows = jnp.where(wid == NW - 1, RPW_LAST, RPW)

    def group(k, carry):
        r = row0 + k * KSUB
        pltpu.sync_copy(src_hbm.at[pl.ds(r, KSUB)], idx_s)
        pltpu.sync_copy(dst_hbm.at[pl.ds(r, KSUB)], idx_d)
        for j in range(KSUB):
            pltpu.async_copy(g_hbm.at[idx_s.at[j]], rows_v.at[j], sem)
        for j in range(KSUB):
            pltpu.make_async_copy(g_hbm.at[idx_s.at[j]], rows_v.at[j], sem).wait()
        for j in range(KSUB):
            pltpu.sync_copy(rows_v.at[j], s_sh.at[idx_d.at[j]], add=True)
        return carry

    lax.fori_loop(0, nrows // KSUB, group, 0)
    plsc.subcore_barrier()

    @pl.when(c == 0)
    def _():
        _stripe_2hop(s, lambda b, n: s_sh.at[pl.ds(b, n)],
                     lambda n: zbuf.at[pl.ds(0, n)],
                     lambda b, n: s_out0.at[pl.ds(b, n)])

    @pl.when(c == 1)
    def _():
        _stripe_2hop(s, lambda b, n: s_sh.at[pl.ds(b, n)],
                     lambda n: zbuf.at[pl.ds(0, n)],
                     lambda b, n: s_out1.at[pl.ds(b, n)])


def _agg_call(g, src_r, dst_r, zeros2):
    mesh = plsc.VectorSubcoreMesh(core_axis_name="c", subcore_axis_name="s")
    return pl.kernel(
        _agg_body,
        out_type=[jax.ShapeDtypeStruct((N, D), jnp.float32),
                  jax.ShapeDtypeStruct((N, D), jnp.float32)],
        mesh=mesh,
        scratch_types=[
            pltpu.VMEM_SHARED((N, D), jnp.float32),
            pltpu.VMEM((KSUB, CH), jnp.int32),
            pltpu.VMEM((KSUB, CH), jnp.int32),
            pltpu.VMEM((KSUB, CH, D), jnp.float32),
            pltpu.VMEM((STRIPE, D), jnp.float32),
            pltpu.SemaphoreType.DMA,
        ],
        compiler_params=pltpu.CompilerParams(use_tc_tiling_on_sc=False),
    )(g, src_r, dst_r, zeros2)


def _scale_body(deg0_ref, deg1_ref, x_ref, g_ref, dinv_ref):
    total = deg0_ref[...] + deg1_ref[...] + 1.0
    dinv = lax.rsqrt(total)
    dinv_ref[...] = dinv
    g_ref[...] = x_ref[...] * dinv


def _scale_call(deg0, deg1, x):
    return pl.pallas_call(
        _scale_body,
        out_shape=[
            jax.ShapeDtypeStruct((N, D), jnp.float32),
            jax.ShapeDtypeStruct((N, 1), jnp.float32),
        ],
    )(deg0, deg1, x)


def _z_body(s0_ref, s1_ref, g_ref, dinv_ref, w_ref, b_ref, z_ref):
    agg = (s0_ref[...] + s1_ref[...] + g_ref[...]) * dinv_ref[...]
    z = jnp.dot(agg, w_ref[...], preferred_element_type=jnp.float32) + b_ref[...]
    z_ref[...] = jnp.maximum(z, 0.0)


def _z_call(s0, s1, g, dinv, W, b2):
    return pl.pallas_call(
        _z_body,
        out_shape=jax.ShapeDtypeStruct((N, D), jnp.float32),
    )(s0, s1, g, dinv, W, b2)


BM = 2000
BN = 2048


def _gram_body(zi_ref, zj_ref, out_ref):
    out_ref[...] = lax.dot_general(
        zi_ref[...], zj_ref[...],
        (((1,), (1,)), ((), ())),
        preferred_element_type=jnp.float32,
    )


def _gram_call(z):
    grid = (N // BM, (N + BN - 1) // BN)
    return pl.pallas_call(
        _gram_body,
        grid=grid,
        in_specs=[
            pl.BlockSpec((BM, D), lambda i, j: (i, 0)),
            pl.BlockSpec((BN, D), lambda i, j: (j, 0)),
        ],
        out_specs=pl.BlockSpec((BM, BN), lambda i, j: (i, j)),
        out_shape=jax.ShapeDtypeStruct((N, N), jnp.float32),
    )(z, z)


def kernel(x, edge_index, W, b):
    src_r = edge_index[0].astype(jnp.int32).reshape(NROW, CH)
    dst_r = edge_index[1].astype(jnp.int32).reshape(NROW, CH)
    ones_c = jnp.ones((CH,), jnp.float32)
    zeros1 = jnp.zeros((STRIPE,), jnp.float32)
    zeros2 = jnp.zeros((STRIPE, D), jnp.float32)

    deg0, deg1 = _deg_call(dst_r, ones_c, zeros1)
    g, dinv = _scale_call(deg0.reshape(N, 1), deg1.reshape(N, 1), x)
    s0, s1 = _agg_call(g, src_r, dst_r, zeros2)
    z = _z_call(s0, s1, g, dinv, W, b.reshape(1, D))
    return _gram_call(z)
